# baseline (speedup 1.0000x reference)
import dataclasses
import functools

import jax
import jax.numpy as jnp
from jax import lax
from jax.experimental import pallas as pl
from jax.experimental.pallas import tpu as pltpu
from jax.experimental.pallas import tpu_sc as plsc

N = 10000
E = 320000
D = 128
NC = 2
NS = 16
NW = NC * NS
EPW = E // NW
C = 80
NCH = EPW // C
CS = 40
NCHS = EPW // CS
NPAD = 10240
ROWS_PT = NPAD // NS
NPADH = 10240
HR = NPADH // 16
HR_PT = HR // NS

_mesh = plsc.VectorSubcoreMesh(core_axis_name="c", subcore_axis_name="s")

_cp = pltpu.CompilerParams()
if "needs_layout_passes" in pltpu.CompilerParams.__dataclass_fields__:
    _cp = dataclasses.replace(_cp, needs_layout_passes=False)
_cp_sg = pltpu.CompilerParams()
if "use_tc_tiling_on_sc" in pltpu.CompilerParams.__dataclass_fields__:
    _cp_sg = dataclasses.replace(_cp_sg, use_tc_tiling_on_sc=False)
    _cp = dataclasses.replace(_cp, use_tc_tiling_on_sc=False)

_f32 = jnp.float32
_i32 = jnp.int32


def _zeros16():
    return jnp.zeros((16,), _f32)


def _hist_body(row_hbm, col_hbm, out_hbm, row_v, col_v, degl, selfl,
               idxr, shd_deg, shd_self):
    cid = lax.axis_index("c")
    sid = lax.axis_index("s")
    wid = sid * NC + cid

    pltpu.sync_copy(row_hbm.at[wid], row_v)
    pltpu.sync_copy(col_hbm.at[wid], col_v)

    @pl.loop(0, HR)
    def _(i):
        degl[i, :] = _zeros16()
        selfl[i, :] = _zeros16()

    pltpu.sync_copy(degl.at[pl.ds(sid * HR_PT, HR_PT)],
                    shd_deg.at[pl.ds(sid * HR_PT, HR_PT)])
    pltpu.sync_copy(selfl.at[pl.ds(sid * HR_PT, HR_PT)],
                    shd_self.at[pl.ds(sid * HR_PT, HR_PT)])

    @pl.loop(0, 5)
    def _(k):
        @pl.loop(0, 8)
        def _(j):
            idxr[k, pl.ds(j * 16, 16)] = (
                lax.iota(_i32, 16) + k * 128 + j * 16)

    ones = jnp.ones((16,), _f32)

    @pl.loop(0, NCH)
    def _(i):
        @pl.loop(0, C // 16)
        def _(j):
            r = row_v[i, pl.ds(j * 16, 16)]
            cc = col_v[i, pl.ds(j * 16, 16)]
            m = r != cc
            hi = lax.shift_right_logical(r, 4)
            lo = lax.bitwise_and(r, 15)
            plsc.addupdate_scatter(degl, [hi, lo], ones, mask=m)
            plsc.addupdate_scatter(selfl, [hi, lo], ones,
                                   mask=jnp.logical_not(m))

    plsc.subcore_barrier()

    @pl.loop(0, 5)
    def _(k):
        pltpu.sync_copy(degl.at[pl.ds(k * 128, 128)],
                        shd_deg.at[idxr.at[k]], add=True)
        pltpu.sync_copy(selfl.at[pl.ds(k * 128, 128)],
                        shd_self.at[idxr.at[k]], add=True)

    plsc.subcore_barrier()

    pltpu.sync_copy(shd_deg.at[pl.ds(sid * HR_PT, HR_PT)],
                    out_hbm.at[cid, 0, pl.ds(sid * HR_PT, HR_PT)])
    pltpu.sync_copy(shd_self.at[pl.ds(sid * HR_PT, HR_PT)],
                    out_hbm.at[cid, 1, pl.ds(sid * HR_PT, HR_PT)])


def _sc_hist(row2d, col2d):
    k = pl.kernel(
        _hist_body,
        out_type=jax.ShapeDtypeStruct((NC, 2, HR, 16), _f32),
        mesh=_mesh,
        scratch_types=[
            pltpu.VMEM((NCH, C), _i32),
            pltpu.VMEM((NCH, C), _i32),
            pltpu.VMEM((HR, 16), _f32),
            pltpu.VMEM((HR, 16), _f32),
            pltpu.VMEM((5, 128), _i32),
            pltpu.VMEM_SHARED((HR, 16), _f32),
            pltpu.VMEM_SHARED((HR, 16), _f32),
        ],
        compiler_params=_cp,
    )
    return k(row2d, col2d)


def _segsum_body(g_hbm, row_hbm, col_hbm, out_hbm, row_v, col_v,
                 gb0, gb1, gb2, gb3, gb4, acc,
                 sg0, sg1, sg2, sg3, sg4, ss0, ss1, ss2, ss3, ss4):
    cid = lax.axis_index("c")
    sid = lax.axis_index("s")
    wid = sid * NC + cid

    gbufs = (gb0, gb1, gb2, gb3, gb4)
    gsems = (sg0, sg1, sg2, sg3, sg4)
    ssems = (ss0, ss1, ss2, ss3, ss4)

    pltpu.async_copy(row_hbm.at[wid], row_v, ss0)
    pltpu.async_copy(col_hbm.at[wid], col_v, sg0)

    def gstart(j, s):
        pltpu.async_copy(g_hbm.at[col_v.at[j]], gbufs[s], gsems[s])

    def gwait(s):
        pltpu.make_async_copy(g_hbm.at[col_v.at[0]], gbufs[s], gsems[s]).wait()

    def sstart(j, s):
        pltpu.async_copy(gbufs[s], acc.at[row_v.at[j]], ssems[s], add=True)

    def swait(s):
        pltpu.make_async_copy(gbufs[s], acc.at[row_v.at[0]], ssems[s]).wait()

    pltpu.make_async_copy(col_hbm.at[wid], col_v, sg0).wait()

    for j in range(4):
        gstart(j, j)

    @pl.loop(0, CS)
    def _(i):
        @pl.loop(0, D // 16)
        def _(j):
            gb4[i, pl.ds(j * 16, 16)] = _zeros16()

    @pl.loop(0, ROWS_PT // (4 * CS))
    def _(w):
        base = sid * ROWS_PT + w * 4 * CS
        for k in range(4):
            pltpu.async_copy(gb4, acc.at[pl.ds(base + k * CS, CS)],
                             ssems[k + 1])
        for k in range(4):
            pltpu.make_async_copy(gb4, acc.at[pl.ds(base + k * CS, CS)],
                                  ssems[k + 1]).wait()

    pltpu.make_async_copy(row_hbm.at[wid], row_v, ss0).wait()

    plsc.subcore_barrier()

    gwait(0)
    sstart(0, 0)
    gstart(4, 4)
    gwait(1)
    sstart(1, 1)

    @pl.loop(5, NCHS, step=5)
    def _(jj):
        for u in range(5):
            swait(u)
            gstart(jj + u, u)
            gwait((u + 2) % 5)
            sstart(jj + u - 3, (u + 2) % 5)

    gwait(2)
    sstart(NCHS - 3, 2)
    gwait(3)
    sstart(NCHS - 2, 3)
    gwait(4)
    sstart(NCHS - 1, 4)
    for u in range(5):
        swait(u)

    plsc.subcore_barrier()

    @pl.loop(0, ROWS_PT // 128)
    def _(i):
        pltpu.sync_copy(acc.at[pl.ds(sid * ROWS_PT + i * 128, 128)],
                        out_hbm.at[cid, pl.ds(sid * ROWS_PT + i * 128, 128)])


def _sc_segsum(g, row2d, col2d):
    k = pl.kernel(
        _segsum_body,
        out_type=jax.ShapeDtypeStruct((NC, NPAD, D), _f32),
        mesh=_mesh,
        scratch_types=[
            pltpu.VMEM((NCHS, CS), _i32),
            pltpu.VMEM((NCHS, CS), _i32),
            pltpu.VMEM((CS, D), _f32),
            pltpu.VMEM((CS, D), _f32),
            pltpu.VMEM((CS, D), _f32),
            pltpu.VMEM((CS, D), _f32),
            pltpu.VMEM((CS, D), _f32),
            pltpu.VMEM_SHARED((NPAD, D), _f32),
        ] + [pltpu.SemaphoreType.DMA] * 10,
        compiler_params=_cp_sg,
    )
    return k(g, row2d, col2d)


def _prep_body(h_ref, x_ref, w0_ref, dis_ref, cc_ref, g1_ref, xw0_ref):
    deg = h_ref[0, 0] + h_ref[1, 0]
    c = h_ref[0, 1] + h_ref[1, 1]
    dis = jnp.where(deg > 0, lax.rsqrt(jnp.maximum(deg, 1e-12)), 0.0)
    dis_ref[...] = dis
    cc_ref[...] = c
    x = x_ref[...]
    g1_ref[0:N, :] = dis[0:N, :] * x
    g1_ref[N:NPAD, :] = jnp.zeros((NPAD - N, D), _f32)
    xw0_ref[...] = jnp.dot(x, w0_ref[...], preferred_element_type=_f32)


def _tc_prep(hist, x, W0):
    return pl.pallas_call(
        _prep_body,
        out_shape=[jax.ShapeDtypeStruct((NPAD, 1), _f32),
                   jax.ShapeDtypeStruct((NPAD, 1), _f32),
                   jax.ShapeDtypeStruct((NPAD, D), _f32),
                   jax.ShapeDtypeStruct((N, D), _f32)],
    )(hist, x, W0)


def _mid_body(s1_ref, g1_ref, dis_ref, cc_ref, w1_ref, g2_ref, t1w1_ref):
    dis = dis_ref[...]
    t1 = -dis * (s1_ref[0] + s1_ref[1] - cc_ref[...] * g1_ref[...])
    g2_ref[...] = dis * t1
    t1w1_ref[...] = jnp.dot(t1[0:N, :], w1_ref[...],
                            preferred_element_type=_f32)


def _tc_mid(s1, g1, dis, cc, W1):
    return pl.pallas_call(
        _mid_body,
        out_shape=[jax.ShapeDtypeStruct((NPAD, D), _f32),
                   jax.ShapeDtypeStruct((N, D), _f32)],
    )(s1, g1, dis, cc, W1)


def _final_body(s2_ref, g2_ref, dis_ref, cc_ref, x_ref, xw0_ref, t1w1_ref,
                w2_ref, b_ref, out_ref):
    t2p = -2.0 * dis_ref[...] * (s2_ref[0] + s2_ref[1]
                                 - cc_ref[...] * g2_ref[...])
    t2 = t2p[0:N, :] - x_ref[...]
    out_ref[...] = (xw0_ref[...] + t1w1_ref[...]
                    + jnp.dot(t2, w2_ref[...], preferred_element_type=_f32)
                    + b_ref[...])


def _tc_final(s2, g2, dis, cc, x, xw0, t1w1, W2, b2d):
    return pl.pallas_call(
        _final_body,
        out_shape=jax.ShapeDtypeStruct((N, D), _f32),
    )(s2, g2, dis, cc, x, xw0, t1w1, W2, b2d)


def kernel(x, edge_index, W0, W1, W2, b):
    row2d = edge_index[0].reshape(NW, NCH, C)
    col2d = edge_index[1].reshape(NW, NCH, C)
    row2s = edge_index[0].reshape(NW, NCHS, CS)
    col2s = edge_index[1].reshape(NW, NCHS, CS)

    hist = _sc_hist(row2d, col2d)
    dis, cc, g1, xw0 = _tc_prep(hist.reshape(NC, 2, NPADH, 1), x, W0)
    s1 = _sc_segsum(g1, row2s, col2s)
    g2, t1w1 = _tc_mid(s1, g1, dis, cc, W1)
    s2 = _sc_segsum(g2, row2s, col2s)
    return _tc_final(s2, g2, dis, cc, x, xw0, t1w1, W2, b.reshape(1, D))


# speedup vs baseline: 1.0680x; 1.0680x over previous
import dataclasses
import functools

import jax
import jax.numpy as jnp
from jax import lax
from jax.experimental import pallas as pl
from jax.experimental.pallas import tpu as pltpu
from jax.experimental.pallas import tpu_sc as plsc

N = 10000
E = 320000
D = 128
NC = 2
NS = 16
NW = NC * NS
EPW = E // NW
C = 80
NCH = EPW // C
CS = 40
NCHS = EPW // CS
NPAD = 10240
ROWS_PT = NPAD // NS
NPADH = 10240
HR = NPADH // 16
HR_PT = HR // NS

_mesh = plsc.VectorSubcoreMesh(core_axis_name="c", subcore_axis_name="s")

_cp = pltpu.CompilerParams()
if "needs_layout_passes" in pltpu.CompilerParams.__dataclass_fields__:
    _cp = dataclasses.replace(_cp, needs_layout_passes=False)
_cp_sg = pltpu.CompilerParams()
if "use_tc_tiling_on_sc" in pltpu.CompilerParams.__dataclass_fields__:
    _cp_sg = dataclasses.replace(_cp_sg, use_tc_tiling_on_sc=False)
    _cp = dataclasses.replace(_cp, use_tc_tiling_on_sc=False)

_f32 = jnp.float32
_i32 = jnp.int32


def _zeros16():
    return jnp.zeros((16,), _f32)


def _hist_body(row_hbm, col_hbm, out_hbm, row_v, col_v, degl, selfl,
               idxr, shd_deg, shd_self):
    cid = lax.axis_index("c")
    sid = lax.axis_index("s")
    wid = sid * NC + cid

    pltpu.sync_copy(row_hbm.at[wid], row_v)
    pltpu.sync_copy(col_hbm.at[wid], col_v)

    @pl.loop(0, HR)
    def _(i):
        degl[i, :] = _zeros16()
        selfl[i, :] = _zeros16()

    pltpu.sync_copy(degl.at[pl.ds(sid * HR_PT, HR_PT)],
                    shd_deg.at[pl.ds(sid * HR_PT, HR_PT)])
    pltpu.sync_copy(selfl.at[pl.ds(sid * HR_PT, HR_PT)],
                    shd_self.at[pl.ds(sid * HR_PT, HR_PT)])

    @pl.loop(0, 5)
    def _(k):
        @pl.loop(0, 8)
        def _(j):
            idxr[k, pl.ds(j * 16, 16)] = (
                lax.iota(_i32, 16) + k * 128 + j * 16)

    ones = jnp.ones((16,), _f32)

    @pl.loop(0, NCH)
    def _(i):
        @pl.loop(0, C // 16)
        def _(j):
            r = row_v[i, pl.ds(j * 16, 16)]
            cc = col_v[i, pl.ds(j * 16, 16)]
            m = r != cc
            hi = lax.shift_right_logical(r, 4)
            lo = lax.bitwise_and(r, 15)
            plsc.addupdate_scatter(degl, [hi, lo], ones, mask=m)
            plsc.addupdate_scatter(selfl, [hi, lo], ones,
                                   mask=jnp.logical_not(m))

    plsc.subcore_barrier()

    @pl.loop(0, 5)
    def _(k):
        pltpu.sync_copy(degl.at[pl.ds(k * 128, 128)],
                        shd_deg.at[idxr.at[k]], add=True)
        pltpu.sync_copy(selfl.at[pl.ds(k * 128, 128)],
                        shd_self.at[idxr.at[k]], add=True)

    plsc.subcore_barrier()

    pltpu.sync_copy(shd_deg.at[pl.ds(sid * HR_PT, HR_PT)],
                    out_hbm.at[cid, 0, pl.ds(sid * HR_PT, HR_PT)])
    pltpu.sync_copy(shd_self.at[pl.ds(sid * HR_PT, HR_PT)],
                    out_hbm.at[cid, 1, pl.ds(sid * HR_PT, HR_PT)])


def _sc_hist(row2d, col2d):
    k = pl.kernel(
        _hist_body,
        out_type=jax.ShapeDtypeStruct((NC, 2, HR, 16), _f32),
        mesh=_mesh,
        scratch_types=[
            pltpu.VMEM((NCH, C), _i32),
            pltpu.VMEM((NCH, C), _i32),
            pltpu.VMEM((HR, 16), _f32),
            pltpu.VMEM((HR, 16), _f32),
            pltpu.VMEM((5, 128), _i32),
            pltpu.VMEM_SHARED((HR, 16), _f32),
            pltpu.VMEM_SHARED((HR, 16), _f32),
        ],
        compiler_params=_cp,
    )
    return k(row2d, col2d)


def _segsum_body(g_hbm, row_hbm, col_hbm, out_hbm, row_v, col_v,
                 gb0, gb1, gb2, gb3, gb4, acc,
                 sg0, sg1, sg2, sg3, sg4, ss0, ss1, ss2, ss3, ss4):
    cid = lax.axis_index("c")
    sid = lax.axis_index("s")
    wid = sid * NC + cid

    gbufs = (gb0, gb1, gb2, gb3, gb4)
    gsems = (sg0, sg1, sg2, sg3, sg4)
    ssems = (ss0, ss1, ss2, ss3, ss4)

    pltpu.async_copy(row_hbm.at[wid], row_v, ss0)
    pltpu.async_copy(col_hbm.at[wid], col_v, sg0)

    def gstart(j, s):
        pltpu.async_copy(g_hbm.at[col_v.at[j]], gbufs[s], gsems[s])

    def gwait(s):
        pltpu.make_async_copy(g_hbm.at[col_v.at[0]], gbufs[s], gsems[s]).wait()

    def sstart(j, s):
        pltpu.async_copy(gbufs[s], acc.at[row_v.at[j]], ssems[s], add=True)

    def swait(s):
        pltpu.make_async_copy(gbufs[s], acc.at[row_v.at[0]], ssems[s]).wait()

    pltpu.make_async_copy(col_hbm.at[wid], col_v, sg0).wait()

    for j in range(4):
        gstart(j, j)

    @pl.loop(0, CS)
    def _(i):
        @pl.loop(0, D // 16)
        def _(j):
            gb4[i, pl.ds(j * 16, 16)] = _zeros16()

    @pl.loop(0, ROWS_PT // (4 * CS))
    def _(w):
        base = sid * ROWS_PT + w * 4 * CS
        for k in range(4):
            pltpu.async_copy(gb4, acc.at[pl.ds(base + k * CS, CS)],
                             ssems[k + 1])
        for k in range(4):
            pltpu.make_async_copy(gb4, acc.at[pl.ds(base + k * CS, CS)],
                                  ssems[k + 1]).wait()

    pltpu.make_async_copy(row_hbm.at[wid], row_v, ss0).wait()

    plsc.subcore_barrier()

    gwait(0)
    sstart(0, 0)
    gstart(4, 4)
    gwait(1)
    sstart(1, 1)

    @pl.loop(5, NCHS, step=5)
    def _(jj):
        for u in range(5):
            swait(u)
            gstart(jj + u, u)
            gwait((u + 2) % 5)
            sstart(jj + u - 3, (u + 2) % 5)

    gwait(2)
    sstart(NCHS - 3, 2)
    gwait(3)
    sstart(NCHS - 2, 3)
    gwait(4)
    sstart(NCHS - 1, 4)
    for u in range(5):
        swait(u)

    plsc.subcore_barrier()

    @pl.loop(0, ROWS_PT // 128)
    def _(i):
        pltpu.sync_copy(acc.at[pl.ds(sid * ROWS_PT + i * 128, 128)],
                        out_hbm.at[cid, pl.ds(sid * ROWS_PT + i * 128, 128)])


def _sc_segsum(g, row2d, col2d):
    k = pl.kernel(
        _segsum_body,
        out_type=jax.ShapeDtypeStruct((NC, NPAD, D), _f32),
        mesh=_mesh,
        scratch_types=[
            pltpu.VMEM((NCHS, CS), _i32),
            pltpu.VMEM((NCHS, CS), _i32),
            pltpu.VMEM((CS, D), _f32),
            pltpu.VMEM((CS, D), _f32),
            pltpu.VMEM((CS, D), _f32),
            pltpu.VMEM((CS, D), _f32),
            pltpu.VMEM((CS, D), _f32),
            pltpu.VMEM_SHARED((NPAD, D), _f32),
        ] + [pltpu.SemaphoreType.DMA] * 10,
        compiler_params=_cp_sg,
    )
    return k(g, row2d, col2d)


def _scales_body(h_ref, sp_ref):
    deg = h_ref[0, 0] + h_ref[1, 0]
    c = h_ref[0, 1] + h_ref[1, 1]
    dis = jnp.where(deg > 0, lax.rsqrt(jnp.maximum(deg, 1e-12)), 0.0)
    sp_ref[0] = dis
    sp_ref[1] = c


def _tc_scales(hist):
    return pl.pallas_call(
        _scales_body,
        out_shape=jax.ShapeDtypeStruct((2, HR, 16), _f32),
    )(hist)


def _g1_body(x_ref, dis_ref, w0_ref, g1_ref, xw0_ref):
    x = x_ref[...]
    g1_ref[0:N, :] = dis_ref[0:N, :] * x
    g1_ref[N:NPAD, :] = jnp.zeros((NPAD - N, D), _f32)
    xw0_ref[...] = jnp.dot(x, w0_ref[...], preferred_element_type=_f32)


def _tc_g1(x, dis, W0):
    return pl.pallas_call(
        _g1_body,
        out_shape=[jax.ShapeDtypeStruct((NPAD, D), _f32),
                   jax.ShapeDtypeStruct((N, D), _f32)],
    )(x, dis, W0)


def _mid_body(s1_ref, g1_ref, dis_ref, cc_ref, w1_ref, g2_ref, t1w1_ref):
    dis = dis_ref[...]
    t1 = -dis * (s1_ref[0] + s1_ref[1] - cc_ref[...] * g1_ref[...])
    g2_ref[...] = dis * t1
    t1w1_ref[...] = jnp.dot(t1[0:N, :], w1_ref[...],
                            preferred_element_type=_f32)


def _tc_mid(s1, g1, dis, cc, W1):
    return pl.pallas_call(
        _mid_body,
        out_shape=[jax.ShapeDtypeStruct((NPAD, D), _f32),
                   jax.ShapeDtypeStruct((N, D), _f32)],
    )(s1, g1, dis, cc, W1)


def _final_body(s2_ref, g2_ref, dis_ref, cc_ref, x_ref, xw0_ref, t1w1_ref,
                w2_ref, b_ref, out_ref):
    t2p = -2.0 * dis_ref[...] * (s2_ref[0] + s2_ref[1]
                                 - cc_ref[...] * g2_ref[...])
    t2 = t2p[0:N, :] - x_ref[...]
    out_ref[...] = (xw0_ref[...] + t1w1_ref[...]
                    + jnp.dot(t2, w2_ref[...], preferred_element_type=_f32)
                    + b_ref[...])


def _tc_final(s2, g2, dis, cc, x, xw0, t1w1, W2, b2d):
    return pl.pallas_call(
        _final_body,
        out_shape=jax.ShapeDtypeStruct((N, D), _f32),
    )(s2, g2, dis, cc, x, xw0, t1w1, W2, b2d)


def kernel(x, edge_index, W0, W1, W2, b):
    row2d = edge_index[0].reshape(NW, NCH, C)
    col2d = edge_index[1].reshape(NW, NCH, C)
    row2s = edge_index[0].reshape(NW, NCHS, CS)
    col2s = edge_index[1].reshape(NW, NCHS, CS)

    hist = _sc_hist(row2d, col2d)
    sp = _tc_scales(hist)
    dis = sp[0].reshape(NPADH, 1)
    cc = sp[1].reshape(NPADH, 1)

    g1, xw0 = _tc_g1(x, dis, W0)
    s1 = _sc_segsum(g1, row2s, col2s)
    g2, t1w1 = _tc_mid(s1, g1, dis, cc, W1)
    s2 = _sc_segsum(g2, row2s, col2s)
    return _tc_final(s2, g2, dis, cc, x, xw0, t1w1, W2, b.reshape(1, D))


# speedup vs baseline: 1.1159x; 1.0449x over previous
import dataclasses
import functools

import jax
import jax.numpy as jnp
from jax import lax
from jax.experimental import pallas as pl
from jax.experimental.pallas import tpu as pltpu
from jax.experimental.pallas import tpu_sc as plsc

N = 10000
E = 320000
D = 128
NC = 2
NS = 16
NW = NC * NS
EPW = E // NW
C = 80
NCH = EPW // C
CS = 40
NCHS = EPW // CS
NPAD = 10240
ROWS_PT = NPAD // NS
NPADH = 10240
HR = NPADH // 16
HR_PT = HR // NS

_mesh = plsc.VectorSubcoreMesh(core_axis_name="c", subcore_axis_name="s")

_cp = pltpu.CompilerParams()
if "needs_layout_passes" in pltpu.CompilerParams.__dataclass_fields__:
    _cp = dataclasses.replace(_cp, needs_layout_passes=False)
_cp_sg = pltpu.CompilerParams()
if "use_tc_tiling_on_sc" in pltpu.CompilerParams.__dataclass_fields__:
    _cp_sg = dataclasses.replace(_cp_sg, use_tc_tiling_on_sc=False)
    _cp = dataclasses.replace(_cp, use_tc_tiling_on_sc=False)

_f32 = jnp.float32
_i32 = jnp.int32


def _zeros16():
    return jnp.zeros((16,), _f32)


def _hist_body(row_hbm, col_hbm, out_hbm, row_v, col_v, degl, selfl,
               idxr, shd_deg, shd_self, sma, smb):
    cid = lax.axis_index("c")
    sid = lax.axis_index("s")
    wid = sid * NC + cid

    pltpu.async_copy(row_hbm.at[wid], row_v, sma)
    pltpu.async_copy(col_hbm.at[wid], col_v, smb)

    @pl.loop(0, HR)
    def _(i):
        degl[i, :] = _zeros16()
        selfl[i, :] = _zeros16()

    pltpu.sync_copy(degl.at[pl.ds(sid * HR_PT, HR_PT)],
                    shd_deg.at[pl.ds(sid * HR_PT, HR_PT)])
    pltpu.sync_copy(selfl.at[pl.ds(sid * HR_PT, HR_PT)],
                    shd_self.at[pl.ds(sid * HR_PT, HR_PT)])

    @pl.loop(0, 5)
    def _(k):
        @pl.loop(0, 8)
        def _(j):
            idxr[k, pl.ds(j * 16, 16)] = (
                lax.iota(_i32, 16) + k * 128 + j * 16)

    ones = jnp.ones((16,), _f32)

    pltpu.make_async_copy(row_hbm.at[wid], row_v, sma).wait()
    pltpu.make_async_copy(col_hbm.at[wid], col_v, smb).wait()

    @pl.loop(0, NCH)
    def _(i):
        @pl.loop(0, C // 16)
        def _(j):
            r = row_v[i, pl.ds(j * 16, 16)]
            cc = col_v[i, pl.ds(j * 16, 16)]
            m = r != cc
            hi = lax.shift_right_logical(r, 4)
            lo = lax.bitwise_and(r, 15)
            plsc.addupdate_scatter(degl, [hi, lo], ones, mask=m)
            plsc.addupdate_scatter(selfl, [hi, lo], ones,
                                   mask=jnp.logical_not(m))

    plsc.subcore_barrier()

    @pl.loop(0, 5)
    def _(k):
        pltpu.sync_copy(degl.at[pl.ds(k * 128, 128)],
                        shd_deg.at[idxr.at[k]], add=True)
        pltpu.sync_copy(selfl.at[pl.ds(k * 128, 128)],
                        shd_self.at[idxr.at[k]], add=True)

    plsc.subcore_barrier()

    pltpu.sync_copy(shd_deg.at[pl.ds(sid * HR_PT, HR_PT)],
                    out_hbm.at[cid, 0, pl.ds(sid * HR_PT, HR_PT)])
    pltpu.sync_copy(shd_self.at[pl.ds(sid * HR_PT, HR_PT)],
                    out_hbm.at[cid, 1, pl.ds(sid * HR_PT, HR_PT)])


def _sc_hist(row2d, col2d):
    k = pl.kernel(
        _hist_body,
        out_type=jax.ShapeDtypeStruct((NC, 2, HR, 16), _f32),
        mesh=_mesh,
        scratch_types=[
            pltpu.VMEM((NCH, C), _i32),
            pltpu.VMEM((NCH, C), _i32),
            pltpu.VMEM((HR, 16), _f32),
            pltpu.VMEM((HR, 16), _f32),
            pltpu.VMEM((5, 128), _i32),
            pltpu.VMEM_SHARED((HR, 16), _f32),
            pltpu.VMEM_SHARED((HR, 16), _f32),
            pltpu.SemaphoreType.DMA,
            pltpu.SemaphoreType.DMA,
        ],
        compiler_params=_cp,
    )
    return k(row2d, col2d)


def _segsum_body(g_hbm, row_hbm, col_hbm, out_hbm, row_v, col_v,
                 gb0, gb1, gb2, gb3, gb4, acc,
                 sg0, sg1, sg2, sg3, sg4, ss0, ss1, ss2, ss3, ss4):
    cid = lax.axis_index("c")
    sid = lax.axis_index("s")
    wid = sid * NC + cid

    gbufs = (gb0, gb1, gb2, gb3, gb4)
    gsems = (sg0, sg1, sg2, sg3, sg4)
    ssems = (ss0, ss1, ss2, ss3, ss4)

    pltpu.async_copy(row_hbm.at[wid], row_v, ss0)
    pltpu.async_copy(col_hbm.at[wid], col_v, sg0)

    def gstart(j, s):
        pltpu.async_copy(g_hbm.at[col_v.at[j]], gbufs[s], gsems[s])

    def gwait(s):
        pltpu.make_async_copy(g_hbm.at[col_v.at[0]], gbufs[s], gsems[s]).wait()

    def sstart(j, s):
        pltpu.async_copy(gbufs[s], acc.at[row_v.at[j]], ssems[s], add=True)

    def swait(s):
        pltpu.make_async_copy(gbufs[s], acc.at[row_v.at[0]], ssems[s]).wait()

    pltpu.make_async_copy(col_hbm.at[wid], col_v, sg0).wait()

    for j in range(4):
        gstart(j, j)

    @pl.loop(0, CS)
    def _(i):
        @pl.loop(0, D // 16)
        def _(j):
            gb4[i, pl.ds(j * 16, 16)] = _zeros16()

    @pl.loop(0, ROWS_PT // (4 * CS))
    def _(w):
        base = sid * ROWS_PT + w * 4 * CS
        for k in range(4):
            pltpu.async_copy(gb4, acc.at[pl.ds(base + k * CS, CS)],
                             ssems[k + 1])
        for k in range(4):
            pltpu.make_async_copy(gb4, acc.at[pl.ds(base + k * CS, CS)],
                                  ssems[k + 1]).wait()

    pltpu.make_async_copy(row_hbm.at[wid], row_v, ss0).wait()

    plsc.subcore_barrier()

    gstart(4, 4)
    gwait(0)
    sstart(0, 0)

    @pl.loop(5, NCHS, step=5)
    def _(jj):
        for u in range(5):
            swait(u)
            gstart(jj + u, u)
            gwait((u + 1) % 5)
            sstart(jj + u - 4, (u + 1) % 5)

    for u in range(4):
        gwait((u + 1) % 5)
        sstart(NCHS - 4 + u, (u + 1) % 5)
    for u in range(5):
        swait(u)

    plsc.subcore_barrier()

    @pl.loop(0, ROWS_PT // 128)
    def _(i):
        pltpu.sync_copy(acc.at[pl.ds(sid * ROWS_PT + i * 128, 128)],
                        out_hbm.at[cid, pl.ds(sid * ROWS_PT + i * 128, 128)])


def _sc_segsum(g, row2d, col2d):
    k = pl.kernel(
        _segsum_body,
        out_type=jax.ShapeDtypeStruct((NC, NPAD, D), _f32),
        mesh=_mesh,
        scratch_types=[
            pltpu.VMEM((NCHS, CS), _i32),
            pltpu.VMEM((NCHS, CS), _i32),
            pltpu.VMEM((CS, D), _f32),
            pltpu.VMEM((CS, D), _f32),
            pltpu.VMEM((CS, D), _f32),
            pltpu.VMEM((CS, D), _f32),
            pltpu.VMEM((CS, D), _f32),
            pltpu.VMEM_SHARED((NPAD, D), _f32),
        ] + [pltpu.SemaphoreType.DMA] * 10,
        compiler_params=_cp_sg,
    )
    return k(g, row2d, col2d)


def _scales_body(h_ref, sp_ref):
    deg = h_ref[0, 0] + h_ref[1, 0]
    c = h_ref[0, 1] + h_ref[1, 1]
    dis = jnp.where(deg > 0, lax.rsqrt(jnp.maximum(deg, 1e-12)), 0.0)
    sp_ref[0] = dis
    sp_ref[1] = c


def _tc_scales(hist):
    return pl.pallas_call(
        _scales_body,
        out_shape=jax.ShapeDtypeStruct((2, HR, 16), _f32),
    )(hist)


def _g1_body(x_ref, dis_ref, w0_ref, g1_ref, xw0_ref):
    x = x_ref[...]
    g1_ref[0:N, :] = dis_ref[0:N, :] * x
    g1_ref[N:NPAD, :] = jnp.zeros((NPAD - N, D), _f32)
    xw0_ref[...] = jnp.dot(x, w0_ref[...], preferred_element_type=_f32)


def _tc_g1(x, dis, W0):
    return pl.pallas_call(
        _g1_body,
        out_shape=[jax.ShapeDtypeStruct((NPAD, D), _f32),
                   jax.ShapeDtypeStruct((N, D), _f32)],
    )(x, dis, W0)


def _mid_body(s1_ref, g1_ref, dis_ref, cc_ref, w1_ref, g2_ref, t1w1_ref):
    dis = dis_ref[...]
    t1 = -dis * (s1_ref[0] + s1_ref[1] - cc_ref[...] * g1_ref[...])
    g2_ref[...] = dis * t1
    t1w1_ref[...] = jnp.dot(t1[0:N, :], w1_ref[...],
                            preferred_element_type=_f32)


def _tc_mid(s1, g1, dis, cc, W1):
    return pl.pallas_call(
        _mid_body,
        out_shape=[jax.ShapeDtypeStruct((NPAD, D), _f32),
                   jax.ShapeDtypeStruct((N, D), _f32)],
    )(s1, g1, dis, cc, W1)


def _final_body(s2_ref, g2_ref, dis_ref, cc_ref, x_ref, xw0_ref, t1w1_ref,
                w2_ref, b_ref, out_ref):
    t2p = -2.0 * dis_ref[...] * (s2_ref[0] + s2_ref[1]
                                 - cc_ref[...] * g2_ref[...])
    t2 = t2p[0:N, :] - x_ref[...]
    out_ref[...] = (xw0_ref[...] + t1w1_ref[...]
                    + jnp.dot(t2, w2_ref[...], preferred_element_type=_f32)
                    + b_ref[...])


def _tc_final(s2, g2, dis, cc, x, xw0, t1w1, W2, b2d):
    return pl.pallas_call(
        _final_body,
        out_shape=jax.ShapeDtypeStruct((N, D), _f32),
    )(s2, g2, dis, cc, x, xw0, t1w1, W2, b2d)


def kernel(x, edge_index, W0, W1, W2, b):
    row2d = edge_index[0].reshape(NW, NCH, C)
    col2d = edge_index[1].reshape(NW, NCH, C)
    row2s = edge_index[0].reshape(NW, NCHS, CS)
    col2s = edge_index[1].reshape(NW, NCHS, CS)

    hist = _sc_hist(row2d, col2d)
    sp = _tc_scales(hist)
    dis = sp[0].reshape(NPADH, 1)
    cc = sp[1].reshape(NPADH, 1)

    g1, xw0 = _tc_g1(x, dis, W0)
    s1 = _sc_segsum(g1, row2s, col2s)
    g2, t1w1 = _tc_mid(s1, g1, dis, cc, W1)
    s2 = _sc_segsum(g2, row2s, col2s)
    return _tc_final(s2, g2, dis, cc, x, xw0, t1w1, W2, b.reshape(1, D))
